# baseline (speedup 1.0000x reference)
import jax
import jax.numpy as jnp
from jax import lax
from jax.experimental import pallas as pl
from jax.experimental.pallas import tpu as pltpu
from jax.experimental.pallas import tpu_sc as plsc

_NC = 2
_NS = 16
_CHUNK = 128
_RING = 3
_AHEAD = _RING - 1


def _make_spmm(n_p, d, chunks, col_split):
    srows = n_p // _NS
    zrows = 32
    zcopies = srows // zrows
    halves = 2 if col_split else 1
    hchunks = chunks // halves
    mesh = plsc.VectorSubcoreMesh(core_axis_name="c", subcore_axis_name="s")

    def body(g_hbm, g_flat, src_hbm, dst_hbm, out_hbm, idxs, idxd, rows,
             zbuf, acc, table, *sems):
        c = lax.axis_index("c")
        s = lax.axis_index("s")
        if col_split:
            sel_s = c * _NS + s
            sel_d = s
        else:
            sel_s = sel_d = s * _NC + c
        rbase = s * srows
        stripe = pl.ds(rbase, srows)

        tsrc = g_hbm.at[c, stripe] if col_split else g_hbm.at[stripe]
        tstage = pltpu.async_copy(tsrc, table.at[stripe], sems[0])

        zv = jnp.zeros((16,), jnp.float32)

        def zrow(r, carry):
            for j in range(d // 16):
                zbuf[r, pl.ds(j * 16, 16)] = zv
            return carry

        lax.fori_loop(0, zrows, zrow, 0)

        for k in range(zcopies):
            pltpu.sync_copy(zbuf, acc.at[pl.ds(rbase + k * zrows, zrows)])
        tstage.wait()
        plsc.subcore_barrier()

        sem_g = sems[:_RING]
        sem_s = sems[_RING:]

        def gather(k, bb):
            src = g_flat if bb == 0 else table
            return pltpu.make_async_copy(
                src.at[idxs.at[k]], rows.at[bb], sem_g[bb])

        def scatter(k, bb):
            return pltpu.make_async_copy(
                rows.at[bb], acc.at[idxd.at[k]], sem_s[bb])

        for h in range(halves):
            pltpu.sync_copy(src_hbm.at[sel_s].at[pl.ds(h * hchunks, hchunks)],
                            idxs)
            pltpu.sync_copy(dst_hbm.at[sel_d].at[pl.ds(h * hchunks, hchunks)],
                            idxd)
            for bb in range(_AHEAD):
                gather(bb, bb).start()

            def outer(j, carry):
                for bb in range(_RING):
                    k = j * _RING + bb
                    gather(k, bb).wait()
                    pltpu.async_copy(rows.at[bb], acc.at[idxd.at[k]],
                                     sem_s[bb], add=True)
                    nb = (bb + _AHEAD) % _RING

                    @pl.when(k + _AHEAD < hchunks)
                    def _():
                        @pl.when(k >= 1)
                        def _():
                            scatter(k - 1, nb).wait()

                        gather(k + _AHEAD, nb).start()
                return carry

            lax.fori_loop(0, hchunks // _RING, outer, 0)
            for m in range(_RING):
                km = hchunks - _RING + m
                scatter(km, km % _RING).wait()
        plsc.subcore_barrier()

        pltpu.sync_copy(acc.at[stripe], out_hbm.at[c].at[stripe])

    return pl.kernel(
        body,
        out_type=jax.ShapeDtypeStruct((_NC, n_p, d), jnp.float32),
        mesh=mesh,
        scratch_types=[
            pltpu.VMEM((hchunks, _CHUNK), jnp.int32),
            pltpu.VMEM((hchunks, _CHUNK), jnp.int32),
            pltpu.VMEM((_RING, _CHUNK, d), jnp.float32),
            pltpu.VMEM((zrows, d), jnp.float32),
            pltpu.VMEM_SHARED((n_p, d), jnp.float32),
            pltpu.VMEM_SHARED((n_p, d), jnp.float32),
        ] + [pltpu.SemaphoreType.DMA] * (2 * _RING),
        compiler_params=pltpu.CompilerParams(use_tc_tiling_on_sc=False),
    )


def _make_deg(n_p, chunks):
    srows = n_p // _NS
    zrows = 32
    zcopies = srows // zrows
    mesh = plsc.VectorSubcoreMesh(core_axis_name="c", subcore_axis_name="s")

    def body(dst_hbm, out_hbm, idxd, ones_b, zbuf, acc, sem, *sem2):
        c = lax.axis_index("c")
        s = lax.axis_index("s")
        sel = s * _NC + c
        rbase = s * srows
        stripe = pl.ds(rbase, srows)

        slab = pltpu.async_copy(dst_hbm.at[sel], idxd, sem)

        zv = jnp.zeros((16,), jnp.float32)
        ov = jnp.ones((16,), jnp.float32)

        def fill(r, carry):
            zbuf[r % zrows, pl.ds(0, 16)] = zv
            ones_b[r, pl.ds(0, 16)] = ov
            return carry

        lax.fori_loop(0, _CHUNK, fill, 0)

        for k in range(zcopies):
            pltpu.sync_copy(zbuf, acc.at[pl.ds(rbase + k * zrows, zrows)])
        slab.wait()
        plsc.subcore_barrier()

        def step(j, carry):
            for m in range(2):
                k = 2 * j + m

                @pl.when(k >= 2)
                def _():
                    pltpu.make_async_copy(ones_b, acc.at[idxd.at[k - 2]],
                                          sem2[m]).wait()

                pltpu.async_copy(ones_b, acc.at[idxd.at[k]], sem2[m],
                                 add=True)
            return carry

        lax.fori_loop(0, chunks // 2, step, 0)
        if chunks % 2:
            k = chunks - 1
            pltpu.make_async_copy(ones_b, acc.at[idxd.at[k - 2]],
                                  sem2[k % 2]).wait()
            pltpu.async_copy(ones_b, acc.at[idxd.at[k]], sem2[k % 2],
                             add=True)
        for kk in (chunks - 2, chunks - 1):
            pltpu.make_async_copy(ones_b, acc.at[idxd.at[kk]],
                                  sem2[kk % 2]).wait()
        plsc.subcore_barrier()

        pltpu.sync_copy(acc.at[stripe], out_hbm.at[c].at[stripe])

    return pl.kernel(
        body,
        out_type=jax.ShapeDtypeStruct((_NC, n_p, 16), jnp.float32),
        mesh=mesh,
        scratch_types=[
            pltpu.VMEM((chunks, _CHUNK), jnp.int32),
            pltpu.VMEM((_CHUNK, 16), jnp.float32),
            pltpu.VMEM((zrows, 16), jnp.float32),
            pltpu.VMEM_SHARED((n_p, 16), jnp.float32),
            pltpu.SemaphoreType.DMA,
            pltpu.SemaphoreType.DMA,
            pltpu.SemaphoreType.DMA,
        ],
        compiler_params=pltpu.CompilerParams(use_tc_tiling_on_sc=False),
    )


_BLK = 1024


def _split_store(o_ref, g):
    half = g.shape[1] // 2
    o_ref[0] = g[:, :half]
    o_ref[1] = g[:, half:]


def _tc_first(x, deg2, W):
    n_p, din = x.shape
    dout = W.shape[1]

    def body(x_ref, deg_ref, w_ref, o_ref, dis_ref):
        dsum = deg_ref[0] + deg_ref[1]
        dis = jnp.where(dsum > 0, lax.rsqrt(jnp.maximum(dsum, 1e-12)), 0.0)
        dis_ref[...] = dis
        d0 = dis[:, 0:1]
        _split_store(o_ref, d0 * jnp.dot(x_ref[...], w_ref[...],
                                         preferred_element_type=jnp.float32))

    return pl.pallas_call(
        body,
        grid=(n_p // _BLK,),
        in_specs=[
            pl.BlockSpec((_BLK, din), lambda i: (i, 0)),
            pl.BlockSpec((_NC, _BLK, 16), lambda i: (0, i, 0)),
            pl.BlockSpec((din, dout), lambda i: (0, 0)),
        ],
        out_specs=[
            pl.BlockSpec((2, _BLK, dout // 2), lambda i: (0, i, 0)),
            pl.BlockSpec((_BLK, 16), lambda i: (i, 0)),
        ],
        out_shape=[
            jax.ShapeDtypeStruct((2, n_p, dout // 2), jnp.float32),
            jax.ShapeDtypeStruct((n_p, 16), jnp.float32),
        ],
    )(x, deg2, W)


def _tc_mid(acc2, dis, W, b, split_out):
    n_p = acc2.shape[1]
    half = acc2.shape[2]
    dout = W.shape[1]

    def body(a_ref, dis_ref, w_ref, b_ref, o_ref):
        d0 = dis_ref[:, 0:1]
        h0 = d0 * a_ref[0] + b_ref[:, :half]
        h1 = d0 * a_ref[1] + b_ref[:, half:]
        h0 = jnp.where(h0 > 0, h0, jnp.exp(jnp.minimum(h0, 0.0)) - 1.0)
        h1 = jnp.where(h1 > 0, h1, jnp.exp(jnp.minimum(h1, 0.0)) - 1.0)
        g = d0 * (jnp.dot(h0, w_ref[:half, :],
                          preferred_element_type=jnp.float32) +
                  jnp.dot(h1, w_ref[half:, :],
                          preferred_element_type=jnp.float32))
        if split_out:
            _split_store(o_ref, g)
        else:
            o_ref[...] = g

    if split_out:
        out_spec = pl.BlockSpec((2, _BLK, dout // 2), lambda i: (0, i, 0))
        out_shape = jax.ShapeDtypeStruct((2, n_p, dout // 2), jnp.float32)
    else:
        out_spec = pl.BlockSpec((_BLK, dout), lambda i: (i, 0))
        out_shape = jax.ShapeDtypeStruct((n_p, dout), jnp.float32)

    return pl.pallas_call(
        body,
        grid=(n_p // _BLK,),
        in_specs=[
            pl.BlockSpec((_NC, _BLK, half), lambda i: (0, i, 0)),
            pl.BlockSpec((_BLK, 16), lambda i: (i, 0)),
            pl.BlockSpec((2 * half, dout), lambda i: (0, 0)),
            pl.BlockSpec((1, 2 * half), lambda i: (0, 0)),
        ],
        out_specs=out_spec,
        out_shape=out_shape,
    )(acc2, dis, W, b)


def _tc_last(acc2, dis, b):
    n_p = acc2.shape[1]
    dout = acc2.shape[2]

    def body(a_ref, dis_ref, b_ref, o_ref):
        d0 = dis_ref[:, 0:1]
        o_ref[...] = d0 * (a_ref[0] + a_ref[1]) + b_ref[...]

    return pl.pallas_call(
        body,
        grid=(n_p // _BLK,),
        in_specs=[
            pl.BlockSpec((_NC, _BLK, dout), lambda i: (0, i, 0)),
            pl.BlockSpec((_BLK, 16), lambda i: (i, 0)),
            pl.BlockSpec((1, dout), lambda i: (0, 0)),
        ],
        out_specs=pl.BlockSpec((_BLK, dout), lambda i: (i, 0)),
        out_shape=jax.ShapeDtypeStruct((n_p, dout), jnp.float32),
    )(acc2, dis, b)


def kernel(x, edge_index, weight, W1, b1, W2, b2, W3, b3, W4, b4, W5, b5,
           W6, b6, W7, b7, W8, b8, W9, b9):
    n, din = x.shape
    e = edge_index.shape[1]
    nw = _NC * _NS

    n_p = -(-(n + 1) // (_NS * _CHUNK)) * (_NS * _CHUNK)
    e_f = e + n
    grain = _RING * _CHUNK
    e_pw = -(-e_f // (nw * grain)) * grain
    pad = e_pw * nw - e_f
    chunks_e = e_pw // _CHUNK
    chunks_c = 2 * chunks_e

    src = edge_index[0]
    dst = edge_index[1]
    loop = jnp.arange(n, dtype=src.dtype)
    srcf = jnp.concatenate([src, loop, jnp.zeros((pad,), src.dtype)])
    dstf = jnp.concatenate([dst, loop, jnp.full((pad,), n, dst.dtype)])
    src_e = srcf.reshape(nw, chunks_e, _CHUNK)
    dst_e = dstf.reshape(nw, chunks_e, _CHUNK)
    src_b = srcf.reshape(_NS, chunks_c, _CHUNK)
    hbm_slot = (jnp.arange(chunks_c, dtype=srcf.dtype) % _RING == 0)
    off = (hbm_slot * n_p)[None, :, None]
    src_c = jnp.concatenate([src_b[None], (src_b + off)[None]])
    src_c = src_c.reshape(2 * _NS, chunks_c, _CHUNK)
    dst_c = dstf.reshape(_NS, chunks_c, _CHUNK)

    xp = jnp.pad(x, ((0, n_p - n), (0, 0)))

    spmm128 = _make_spmm(n_p, 64, chunks_c, col_split=True)
    spmm64 = _make_spmm(n_p, 64, chunks_e, col_split=False)

    deg2 = _make_deg(n_p, chunks_e)(dst_e)

    w9p = jnp.pad(W9, ((0, 0), (0, 64 - W9.shape[1])))
    b9p = jnp.pad(b9, (0, 64 - b9.shape[0])).reshape(1, 64)
    mats = [W2, W3, W4, W5, W6, W7, W8, w9p]
    biases = [b1, b2, b3, b4, b5, b6, b7, b8]

    g2, dis = _tc_first(xp, deg2, W1)
    for i in range(8):
        acc2 = spmm128(g2, g2.reshape(2 * n_p, 64), src_c, dst_c)
        g2 = _tc_mid(acc2, dis, mats[i], biases[i].reshape(1, -1),
                     split_out=(i < 7))
    acc2 = spmm64(g2, g2, src_e, dst_e)
    y = _tc_last(acc2, dis, b9p)
    return y[:n, :W9.shape[1]]


# speedup vs baseline: 1.3619x; 1.3619x over previous
import jax
import jax.numpy as jnp
from jax import lax
from jax.experimental import pallas as pl
from jax.experimental.pallas import tpu as pltpu
from jax.experimental.pallas import tpu_sc as plsc

_NC = 2
_NS = 16
_CHUNK = 128
_RING = 3
_AHEAD = _RING - 1


def _make_spmm(n_p, d, chunks, col_split):
    srows = n_p // _NS
    zrows = 32
    zcopies = srows // zrows
    halves = 2 if col_split else 1
    hchunks = chunks // halves
    mesh = plsc.VectorSubcoreMesh(core_axis_name="c", subcore_axis_name="s")

    def body(g_hbm, src_hbm, dst_hbm, out_hbm, idxs, idxd, rows,
             zbuf, acc, table, *sems):
        c = lax.axis_index("c")
        s = lax.axis_index("s")
        if col_split:
            sel_s = sel_d = s
        else:
            sel_s = sel_d = s * _NC + c
        rbase = s * srows
        stripe = pl.ds(rbase, srows)

        tsrc = g_hbm.at[c, stripe] if col_split else g_hbm.at[stripe]
        tstage = pltpu.async_copy(tsrc, table.at[stripe], sems[0])

        zv = jnp.zeros((16,), jnp.float32)

        def zrow(r, carry):
            for j in range(d // 16):
                zbuf[r, pl.ds(j * 16, 16)] = zv
            return carry

        lax.fori_loop(0, zrows, zrow, 0)

        for k in range(zcopies):
            pltpu.sync_copy(zbuf, acc.at[pl.ds(rbase + k * zrows, zrows)])
        tstage.wait()
        plsc.subcore_barrier()

        sem_g = sems[:_RING]
        sem_s = sems[_RING:]

        def gather(k, bb):
            return pltpu.make_async_copy(
                table.at[idxs.at[k]], rows.at[bb], sem_g[bb])

        def scatter(k, bb):
            return pltpu.make_async_copy(
                rows.at[bb], acc.at[idxd.at[k]], sem_s[bb])

        for h in range(halves):
            pltpu.sync_copy(src_hbm.at[sel_s].at[pl.ds(h * hchunks, hchunks)],
                            idxs)
            pltpu.sync_copy(dst_hbm.at[sel_d].at[pl.ds(h * hchunks, hchunks)],
                            idxd)
            for bb in range(_AHEAD):
                gather(bb, bb).start()

            def outer(j, carry):
                for bb in range(_RING):
                    k = j * _RING + bb
                    gather(k, bb).wait()
                    pltpu.async_copy(rows.at[bb], acc.at[idxd.at[k]],
                                     sem_s[bb], add=True)
                    nb = (bb + _AHEAD) % _RING

                    @pl.when(k + _AHEAD < hchunks)
                    def _():
                        @pl.when(k >= 1)
                        def _():
                            scatter(k - 1, nb).wait()

                        gather(k + _AHEAD, nb).start()
                return carry

            lax.fori_loop(0, hchunks // _RING, outer, 0)
            for m in range(_RING):
                km = hchunks - _RING + m
                scatter(km, km % _RING).wait()
        plsc.subcore_barrier()

        pltpu.sync_copy(acc.at[stripe], out_hbm.at[c].at[stripe])

    return pl.kernel(
        body,
        out_type=jax.ShapeDtypeStruct((_NC, n_p, d), jnp.float32),
        mesh=mesh,
        scratch_types=[
            pltpu.VMEM((hchunks, _CHUNK), jnp.int32),
            pltpu.VMEM((hchunks, _CHUNK), jnp.int32),
            pltpu.VMEM((_RING, _CHUNK, d), jnp.float32),
            pltpu.VMEM((zrows, d), jnp.float32),
            pltpu.VMEM_SHARED((n_p, d), jnp.float32),
            pltpu.VMEM_SHARED((n_p, d), jnp.float32),
        ] + [pltpu.SemaphoreType.DMA] * (2 * _RING),
        compiler_params=pltpu.CompilerParams(use_tc_tiling_on_sc=False),
    )


def _make_deg(n_p, chunks):
    srows = n_p // _NS
    zrows = 32
    zcopies = srows // zrows
    mesh = plsc.VectorSubcoreMesh(core_axis_name="c", subcore_axis_name="s")

    def body(dst_hbm, out_hbm, idxd, ones_b, zbuf, acc, sem, *sem2):
        c = lax.axis_index("c")
        s = lax.axis_index("s")
        sel = s * _NC + c
        rbase = s * srows
        stripe = pl.ds(rbase, srows)

        slab = pltpu.async_copy(dst_hbm.at[sel], idxd, sem)

        zv = jnp.zeros((16,), jnp.float32)
        ov = jnp.ones((16,), jnp.float32)

        def fill(r, carry):
            zbuf[r % zrows, pl.ds(0, 16)] = zv
            ones_b[r, pl.ds(0, 16)] = ov
            return carry

        lax.fori_loop(0, _CHUNK, fill, 0)

        for k in range(zcopies):
            pltpu.sync_copy(zbuf, acc.at[pl.ds(rbase + k * zrows, zrows)])
        slab.wait()
        plsc.subcore_barrier()

        def step(j, carry):
            for m in range(2):
                k = 2 * j + m

                @pl.when(k >= 2)
                def _():
                    pltpu.make_async_copy(ones_b, acc.at[idxd.at[k - 2]],
                                          sem2[m]).wait()

                pltpu.async_copy(ones_b, acc.at[idxd.at[k]], sem2[m],
                                 add=True)
            return carry

        lax.fori_loop(0, chunks // 2, step, 0)
        if chunks % 2:
            k = chunks - 1
            pltpu.make_async_copy(ones_b, acc.at[idxd.at[k - 2]],
                                  sem2[k % 2]).wait()
            pltpu.async_copy(ones_b, acc.at[idxd.at[k]], sem2[k % 2],
                             add=True)
        for kk in (chunks - 2, chunks - 1):
            pltpu.make_async_copy(ones_b, acc.at[idxd.at[kk]],
                                  sem2[kk % 2]).wait()
        plsc.subcore_barrier()

        pltpu.sync_copy(acc.at[stripe], out_hbm.at[c].at[stripe])

    return pl.kernel(
        body,
        out_type=jax.ShapeDtypeStruct((_NC, n_p, 16), jnp.float32),
        mesh=mesh,
        scratch_types=[
            pltpu.VMEM((chunks, _CHUNK), jnp.int32),
            pltpu.VMEM((_CHUNK, 16), jnp.float32),
            pltpu.VMEM((zrows, 16), jnp.float32),
            pltpu.VMEM_SHARED((n_p, 16), jnp.float32),
            pltpu.SemaphoreType.DMA,
            pltpu.SemaphoreType.DMA,
            pltpu.SemaphoreType.DMA,
        ],
        compiler_params=pltpu.CompilerParams(use_tc_tiling_on_sc=False),
    )


_BLK = 1024


def _split_store(o_ref, g):
    half = g.shape[1] // 2
    o_ref[0] = g[:, :half]
    o_ref[1] = g[:, half:]


def _tc_first(x, deg2, W):
    n_p, din = x.shape
    dout = W.shape[1]

    def body(x_ref, deg_ref, w_ref, o_ref, dis_ref):
        dsum = deg_ref[0] + deg_ref[1]
        dis = jnp.where(dsum > 0, lax.rsqrt(jnp.maximum(dsum, 1e-12)), 0.0)
        dis_ref[...] = dis
        d0 = dis[:, 0:1]
        _split_store(o_ref, d0 * jnp.dot(x_ref[...], w_ref[...],
                                         preferred_element_type=jnp.float32))

    return pl.pallas_call(
        body,
        grid=(n_p // _BLK,),
        in_specs=[
            pl.BlockSpec((_BLK, din), lambda i: (i, 0)),
            pl.BlockSpec((_NC, _BLK, 16), lambda i: (0, i, 0)),
            pl.BlockSpec((din, dout), lambda i: (0, 0)),
        ],
        out_specs=[
            pl.BlockSpec((2, _BLK, dout // 2), lambda i: (0, i, 0)),
            pl.BlockSpec((_BLK, 16), lambda i: (i, 0)),
        ],
        out_shape=[
            jax.ShapeDtypeStruct((2, n_p, dout // 2), jnp.float32),
            jax.ShapeDtypeStruct((n_p, 16), jnp.float32),
        ],
    )(x, deg2, W)


def _tc_mid(acc2, dis, W, b, split_out):
    n_p = acc2.shape[1]
    half = acc2.shape[2]
    dout = W.shape[1]

    def body(a_ref, dis_ref, w_ref, b_ref, o_ref):
        d0 = dis_ref[:, 0:1]
        h0 = d0 * a_ref[0] + b_ref[:, :half]
        h1 = d0 * a_ref[1] + b_ref[:, half:]
        h0 = jnp.where(h0 > 0, h0, jnp.exp(jnp.minimum(h0, 0.0)) - 1.0)
        h1 = jnp.where(h1 > 0, h1, jnp.exp(jnp.minimum(h1, 0.0)) - 1.0)
        g = d0 * (jnp.dot(h0, w_ref[:half, :],
                          preferred_element_type=jnp.float32) +
                  jnp.dot(h1, w_ref[half:, :],
                          preferred_element_type=jnp.float32))
        if split_out:
            _split_store(o_ref, g)
        else:
            o_ref[...] = g

    if split_out:
        out_spec = pl.BlockSpec((2, _BLK, dout // 2), lambda i: (0, i, 0))
        out_shape = jax.ShapeDtypeStruct((2, n_p, dout // 2), jnp.float32)
    else:
        out_spec = pl.BlockSpec((_BLK, dout), lambda i: (i, 0))
        out_shape = jax.ShapeDtypeStruct((n_p, dout), jnp.float32)

    return pl.pallas_call(
        body,
        grid=(n_p // _BLK,),
        in_specs=[
            pl.BlockSpec((_NC, _BLK, half), lambda i: (0, i, 0)),
            pl.BlockSpec((_BLK, 16), lambda i: (i, 0)),
            pl.BlockSpec((2 * half, dout), lambda i: (0, 0)),
            pl.BlockSpec((1, 2 * half), lambda i: (0, 0)),
        ],
        out_specs=out_spec,
        out_shape=out_shape,
    )(acc2, dis, W, b)


def _tc_last(acc2, dis, b):
    n_p = acc2.shape[1]
    dout = acc2.shape[2]

    def body(a_ref, dis_ref, b_ref, o_ref):
        d0 = dis_ref[:, 0:1]
        o_ref[...] = d0 * (a_ref[0] + a_ref[1]) + b_ref[...]

    return pl.pallas_call(
        body,
        grid=(n_p // _BLK,),
        in_specs=[
            pl.BlockSpec((_NC, _BLK, dout), lambda i: (0, i, 0)),
            pl.BlockSpec((_BLK, 16), lambda i: (i, 0)),
            pl.BlockSpec((1, dout), lambda i: (0, 0)),
        ],
        out_specs=pl.BlockSpec((_BLK, dout), lambda i: (i, 0)),
        out_shape=jax.ShapeDtypeStruct((n_p, dout), jnp.float32),
    )(acc2, dis, b)


def kernel(x, edge_index, weight, W1, b1, W2, b2, W3, b3, W4, b4, W5, b5,
           W6, b6, W7, b7, W8, b8, W9, b9):
    n, din = x.shape
    e = edge_index.shape[1]
    nw = _NC * _NS

    n_p = -(-(n + 1) // (_NS * _CHUNK)) * (_NS * _CHUNK)
    e_f = e + n
    grain = _RING * _CHUNK
    e_pw = -(-e_f // (nw * grain)) * grain
    pad = e_pw * nw - e_f
    chunks_e = e_pw // _CHUNK
    chunks_c = 2 * chunks_e

    src = edge_index[0]
    dst = edge_index[1]
    loop = jnp.arange(n, dtype=src.dtype)
    srcf = jnp.concatenate([src, loop, jnp.zeros((pad,), src.dtype)])
    dstf = jnp.concatenate([dst, loop, jnp.full((pad,), n, dst.dtype)])
    src_e = srcf.reshape(nw, chunks_e, _CHUNK)
    dst_e = dstf.reshape(nw, chunks_e, _CHUNK)
    src_c = srcf.reshape(_NS, chunks_c, _CHUNK)
    dst_c = dstf.reshape(_NS, chunks_c, _CHUNK)

    xp = jnp.pad(x, ((0, n_p - n), (0, 0)))

    spmm128 = _make_spmm(n_p, 64, chunks_c, col_split=True)
    spmm64 = _make_spmm(n_p, 64, chunks_e, col_split=False)

    deg2 = _make_deg(n_p, chunks_e)(dst_e)

    w9p = jnp.pad(W9, ((0, 0), (0, 64 - W9.shape[1])))
    b9p = jnp.pad(b9, (0, 64 - b9.shape[0])).reshape(1, 64)
    mats = [W2, W3, W4, W5, W6, W7, W8, w9p]
    biases = [b1, b2, b3, b4, b5, b6, b7, b8]

    g2, dis = _tc_first(xp, deg2, W1)
    for i in range(8):
        acc2 = spmm128(g2, src_c, dst_c)
        g2 = _tc_mid(acc2, dis, mats[i], biases[i].reshape(1, -1),
                     split_out=(i < 7))
    acc2 = spmm64(g2, src_e, dst_e)
    y = _tc_last(acc2, dis, b9p)
    return y[:n, :W9.shape[1]]
